# baseline (speedup 1.0000x reference)
import functools

import jax
import jax.numpy as jnp
from jax import lax
from jax.experimental import pallas as pl
from jax.experimental.pallas import tpu as pltpu
from jax.experimental.pallas import tpu_sc as plsc

NUM_CLASSES = 16
THR = 0.05
TOPK = 20000
P_TOTAL = 256 * 256
NS = 16
L = 16
PP = P_TOTAL // NS
NCHUNK = 8
CHUNK = PP // NCHUNK
NB = 1024

_mesh = plsc.VectorSubcoreMesh(core_axis_name="c", subcore_axis_name="s")


def _iota():
  return lax.iota(jnp.int32, L)


def _xt_i(v, lane):
  return jnp.sum(jnp.where(_iota() == lane, v, jnp.zeros((L,), jnp.int32)))


@functools.partial(
    pl.kernel,
    out_type=jax.ShapeDtypeStruct((L,), jnp.float32),
    mesh=_mesh,
    compiler_params=pltpu.CompilerParams(needs_layout_passes=False),
    scratch_types=[
        pltpu.VMEM((2, 2, NUM_CLASSES, CHUNK), jnp.float32),
        pltpu.SemaphoreType.DMA,
        pltpu.SemaphoreType.DMA,
        pltpu.VMEM((NUM_CLASSES * PP,), jnp.float32),
        pltpu.VMEM((NB,), jnp.int32),
        pltpu.VMEM((NS, NB), jnp.int32),
        pltpu.VMEM((L,), jnp.int32),
        pltpu.VMEM((L,), jnp.float32),
        pltpu.VMEM((NS, 128), jnp.float32),
        pltpu.VMEM((128,), jnp.float32),
        pltpu.VMEM_SHARED((NS, NB), jnp.int32),
        pltpu.VMEM_SHARED((L,), jnp.int32),
        pltpu.VMEM_SHARED((L,), jnp.float32),
        pltpu.VMEM_SHARED((NS, 128), jnp.float32),
    ],
)
def _sc_kernel(in_hbm, out_hbm, stage, sem0, sem1, scores, hist, hist_all, ctli, ctlf,
               sums_all, fvec, sh_hist, sh_ctli, sh_ctlf, sh_sums):
  tid = lax.axis_index("s")
  cid = lax.axis_index("c")
  iota = _iota()
  ones_i = jnp.ones((L,), jnp.int32)
  zeros_i = jnp.zeros((L,), jnp.int32)

  def zero_hist():
    @plsc.parallel_loop(0, NB, L, unroll=4)
    def _(o):
      hist[pl.ds(o, L)] = zeros_i

  zero_hist()

  thr = jnp.float32(THR)
  sems = (sem0, sem1)

  def _copy(g, par):
    return pltpu.make_async_copy(
        in_hbm.at[:, :, tid * NCHUNK + g], stage.at[par], sems[par])

  _copy(0, 0).start()
  for g in range(NCHUNK):
    par = g & 1
    _copy(g, par).wait()
    if g + 1 < NCHUNK:
      _copy(g + 1, par ^ 1).start()

    @plsc.parallel_loop(0, NUM_CLASSES * CHUNK, L, unroll=4)
    def _(o, g=g, par=par):
      c = lax.shift_right_logical(o, 9)
      off = o & (CHUNK - 1)
      a = stage[par, 0, c, pl.ds(off, L)]
      b = stage[par, 1, c, pl.ds(off, L)]
      sa = 1.0 / (1.0 + jnp.exp(-a))
      sb = 1.0 / (1.0 + jnp.exp(-b))
      s = 0.5 * (sa + sb)
      scores[pl.ds(c * PP + g * CHUNK + off, L)] = s
      bits = lax.bitcast_convert_type(s, jnp.int32)
      vmask = s > thr
      bkt = lax.shift_right_logical(bits, 20)
      plsc.addupdate_scatter(hist, [bkt], ones_i, mask=vmask)

  def scan_hist(k_rank):
    def sbody(jj, carry):
      r, b_star, rem = carry
      j = (NB // L - 1) - jj
      m = zeros_i
      for t in range(NS):
        m = m + hist_all[t, pl.ds(j * L, L)]
      rm = lax.rev(m, (0,))
      cs = plsc.cumsum(rm)
      tot = jnp.sum(m)
      ge = (r + cs) >= k_rank
      ffsv = plsc.all_reduce_ffs(ge)
      i = _xt_i(ffsv, 0)
      lane = (L - 1) - i
      bucket = j * L + lane
      mb = jnp.sum(jnp.where(iota == i, rm, zeros_i))
      csi = jnp.sum(jnp.where(iota == i, cs, zeros_i))
      cum_before = r + csi - mb
      hit = (r < k_rank) & ((r + tot) >= k_rank)
      b_star = jnp.where(hit, bucket, b_star)
      rem = jnp.where(hit, k_rank - cum_before, rem)
      return (r + tot, b_star, rem)
    r, b_star, rem = lax.fori_loop(
        0, NB // L, sbody,
        (jnp.int32(0), jnp.int32(-1), jnp.int32(0)))
    return b_star, rem, r

  def publish_hist():
    pltpu.sync_copy(hist, sh_hist.at[tid])
    plsc.subcore_barrier()

  def read_ctl():
    pltpu.sync_copy(sh_ctli, ctli)
    v = ctli[pl.ds(0, L)]
    return _xt_i(v, 0), _xt_i(v, 1), _xt_i(v, 2), _xt_i(v, 3)

  def write_ctl(p, k, ok, nv):
    v = (jnp.where(iota == 0, p, 0) + jnp.where(iota == 1, k, 0)
         + jnp.where(iota == 2, ok, 0) + jnp.where(iota == 3, nv, 0))
    ctli[pl.ds(0, L)] = v.astype(jnp.int32)
    pltpu.sync_copy(ctli, sh_ctli)

  publish_hist()

  @pl.when(tid == 0)
  def _():
    pltpu.sync_copy(sh_hist, hist_all)
    b1, rem1, nv = scan_hist(jnp.int32(TOPK))
    ok = (b1 >= 0).astype(jnp.int32)
    write_ctl(lax.shift_left(jnp.maximum(b1, 0), 20), rem1, ok, nv)

  zero_hist()
  plsc.subcore_barrier()

  for shift in (10, 0):
    p_run, k_run, ok_run, nv_run = read_ctl()
    hi = shift + 10

    @plsc.parallel_loop(0, NUM_CLASSES * PP, L, unroll=8)
    def _(o, shift=shift, hi=hi, p_run=p_run):
      s = scores[pl.ds(o, L)]
      bits = lax.bitcast_convert_type(s, jnp.int32)
      m = (s > thr) & (lax.shift_right_logical(bits, hi)
                       == lax.shift_right_logical(p_run, hi))
      bkt = lax.shift_right_logical(bits, shift) & (NB - 1)
      plsc.addupdate_scatter(hist, [bkt], ones_i, mask=m)

    publish_hist()

    @pl.when(tid == 0)
    def _(shift=shift, p_run=p_run, k_run=k_run, ok_run=ok_run,
          nv_run=nv_run):
      pltpu.sync_copy(sh_hist, hist_all)
      b, rem, _tot = scan_hist(k_run)
      ok = ok_run * (b >= 0).astype(jnp.int32)
      p_new = p_run | lax.shift_left(jnp.maximum(b, 0), shift)
      write_ctl(p_new, rem, ok, nv_run)
      if shift == 0:
        strict = jnp.where((nv_run < TOPK) | (ok == 0),
                           jnp.int32(1), jnp.int32(0))
        t_bits = jnp.zeros((L,), jnp.int32) + p_new
        t_vec = lax.bitcast_convert_type(t_bits, jnp.float32)
        t_vec = jnp.where((jnp.zeros((L,), jnp.int32) + strict) == 1,
                          jnp.full((L,), THR, jnp.float32), t_vec)
        ctlf[pl.ds(0, L)] = t_vec
        pltpu.sync_copy(ctlf, sh_ctlf)
        v = (jnp.where(iota == 0, p_new, 0) + jnp.where(iota == 1, rem, 0)
             + jnp.where(iota == 2, strict, 0)
             + jnp.where(iota == 3, nv_run, 0))
        ctli[pl.ds(0, L)] = v.astype(jnp.int32)
        pltpu.sync_copy(ctli, sh_ctli)

    if shift != 0:
      zero_hist()
    plsc.subcore_barrier()

  _p_fin, _k_fin, strict_fin, _nv = read_ctl()
  pltpu.sync_copy(sh_ctlf, ctlf)
  t_vec = ctlf[pl.ds(0, L)]
  nonstrict = (jnp.zeros((L,), jnp.int32) + strict_fin) == 0

  zf = jnp.zeros((L,), jnp.float32)

  @plsc.parallel_loop(0, PP, L, unroll=4,
                      carry=(zf, zf, zf, zf, zf, zf, zf, zf, zf, zf))
  def acc(base, carry):
    nX, SxX, SyX, SxyX, SxxX, nY, SxY, SyY, SxyY, SxxY = carry
    kept = jnp.zeros((L,), jnp.bool_)
    xv = None
    yv = None
    for c in range(NUM_CLASSES):
      sv = scores[pl.ds(c * PP + base, L)]
      selc = (sv > t_vec) | (nonstrict & (sv == t_vec))
      kept = kept | selc
      if c == 3:
        xv = sv
      if c == 5:
        yv = sv
    gt = xv > yv
    mx = kept & gt
    my = kept & (~gt)
    x0 = xv - 0.5
    y0 = yv - 0.5
    wx = jnp.where(mx, 1.0, 0.0).astype(jnp.float32)
    wy = jnp.where(my, 1.0, 0.0).astype(jnp.float32)
    xy = x0 * y0
    xx = x0 * x0
    return (nX + wx, SxX + x0 * wx, SyX + y0 * wx, SxyX + xy * wx,
            SxxX + xx * wx, nY + wy, SxY + x0 * wy, SyY + y0 * wy,
            SxyY + xy * wy, SxxY + xx * wy)
  sums = zf
  for i, a in enumerate(acc):
    sums = sums + jnp.where(iota == i, jnp.sum(a), jnp.float32(0.0))
  fvec[pl.ds(0, L)] = sums
  pltpu.sync_copy(fvec, sh_sums.at[tid])
  plsc.subcore_barrier()

  @pl.when((tid == 0) & (cid == 0))
  def _():
    pltpu.sync_copy(sh_sums, sums_all)
    tv = zf
    for t in range(NS):
      tv = tv + sums_all[t, pl.ds(0, L)]

    def xt(lane):
      return jnp.zeros((L,), jnp.float32) + jnp.sum(
          jnp.where(iota == lane, tv, zf))

    nX, SxX, SyX, SxyX, SxxX = xt(0), xt(1), xt(2), xt(3), xt(4)
    nY, SxY, SyY, SxyY, SxxY = xt(5), xt(6), xt(7), xt(8), xt(9)

    def slope(n, sx, sy, sxy, sxx):
      return (sxy - sx * sy / n) / (sxx - sx * sx / n)

    sx_ = slope(nX, SxX, SyX, SxyX, SxxX)
    sy_ = slope(nY, SxY, SyY, SxyY, SxxY)
    t = jnp.abs((sy_ - sx_) / (1.0 + sy_ * sx_))
    tan3pi8 = jnp.float32(2.414213562373095)
    tanpi8 = jnp.float32(0.4142135623730951)
    big = t > tan3pi8
    mid = (t > tanpi8) & (~big)
    u = jnp.where(big, -1.0 / t, jnp.where(mid, (t - 1.0) / (t + 1.0), t))
    z = u * u
    poly = ((jnp.float32(8.05374449538e-2) * z
             - jnp.float32(1.38776856032e-1)) * z
            + jnp.float32(1.99777106478e-1)) * z - jnp.float32(
                3.33329491539e-1)
    at = poly * z * u + u
    at = at + jnp.where(big, jnp.float32(1.5707963267948966),
                        jnp.where(mid, jnp.float32(0.7853981633974483),
                                  jnp.float32(0.0)))
    sca = at * jnp.float32(57.29577951308232)
    nkept = nX + nY
    res = jnp.where(nkept > 0.5, sca, jnp.full((L,), jnp.nan, jnp.float32))
    fvec[pl.ds(0, L)] = res
    pltpu.sync_copy(fvec.at[pl.ds(0, L)], out_hbm)


def kernel(cls_score):
  x = cls_score.reshape(2, NUM_CLASSES, NS * NCHUNK, CHUNK)
  out = _sc_kernel(x)
  return out[0]


# speedup vs baseline: 1.0962x; 1.0962x over previous
import functools

import jax
import jax.numpy as jnp
from jax import lax
from jax.experimental import pallas as pl
from jax.experimental.pallas import tpu as pltpu
from jax.experimental.pallas import tpu_sc as plsc

NUM_CLASSES = 16
THR = 0.05
TOPK = 20000
P_TOTAL = 256 * 256
NS = 16
L = 16
PP = P_TOTAL // NS
NCHUNK = 8
CHUNK = PP // NCHUNK
NB = 1024

_mesh = plsc.VectorSubcoreMesh(core_axis_name="c", subcore_axis_name="s")


def _iota():
  return lax.iota(jnp.int32, L)


def _xt_i(v, lane):
  return jnp.sum(jnp.where(_iota() == lane, v, jnp.zeros((L,), jnp.int32)))


@functools.partial(
    pl.kernel,
    out_type=jax.ShapeDtypeStruct((L,), jnp.float32),
    mesh=_mesh,
    compiler_params=pltpu.CompilerParams(needs_layout_passes=False),
    scratch_types=[
        pltpu.VMEM((2, 2, NUM_CLASSES, CHUNK), jnp.float32),
        pltpu.SemaphoreType.DMA,
        pltpu.SemaphoreType.DMA,
        pltpu.VMEM((NUM_CLASSES * PP,), jnp.float32),
        pltpu.VMEM((NB,), jnp.int32),
        pltpu.VMEM((NS, NB), jnp.int32),
        pltpu.VMEM((L,), jnp.int32),
        pltpu.VMEM((L,), jnp.float32),
        pltpu.VMEM((NS, 128), jnp.float32),
        pltpu.VMEM((128,), jnp.float32),
        pltpu.VMEM_SHARED((NS, NB), jnp.int32),
        pltpu.VMEM_SHARED((L,), jnp.int32),
        pltpu.VMEM_SHARED((L,), jnp.float32),
        pltpu.VMEM_SHARED((NS, 128), jnp.float32),
    ],
)
def _sc_kernel(in_hbm, out_hbm, stage, sem0, sem1, scores, hist, hist_all, ctli, ctlf,
               sums_all, fvec, sh_hist, sh_ctli, sh_ctlf, sh_sums):
  tid = lax.axis_index("s")
  cid = lax.axis_index("c")
  iota = _iota()
  ones_i = jnp.ones((L,), jnp.int32)
  zeros_i = jnp.zeros((L,), jnp.int32)

  def zero_hist():
    @plsc.parallel_loop(0, NB, L, unroll=4)
    def _(o):
      hist[pl.ds(o, L)] = zeros_i

  zero_hist()

  thr = jnp.float32(THR)
  sems = (sem0, sem1)

  def _copy(g, par):
    return pltpu.make_async_copy(
        in_hbm.at[:, :, tid * NCHUNK + g], stage.at[par], sems[par])

  _copy(0, 0).start()
  for g in range(NCHUNK):
    par = g & 1
    _copy(g, par).wait()
    if g + 1 < NCHUNK:
      _copy(g + 1, par ^ 1).start()

    @plsc.parallel_loop(0, NUM_CLASSES * CHUNK, L, unroll=2)
    def _(o, g=g, par=par):
      c = lax.shift_right_logical(o, 9)
      off = o & (CHUNK - 1)
      a = stage[par, 0, c, pl.ds(off, L)]
      b = stage[par, 1, c, pl.ds(off, L)]
      sa = 1.0 / (1.0 + jnp.exp(-a))
      sb = 1.0 / (1.0 + jnp.exp(-b))
      s = 0.5 * (sa + sb)
      scores[pl.ds(c * PP + g * CHUNK + off, L)] = s
      bits = lax.bitcast_convert_type(s, jnp.int32)
      vmask = s > thr
      bkt = lax.shift_right_logical(bits, 20)
      plsc.addupdate_scatter(hist, [bkt], ones_i, mask=vmask)

  def scan_hist(k_rank):
    def sbody(jj, carry):
      r, b_star, rem = carry
      j = (NB // L - 1) - jj
      m = zeros_i
      for t in range(NS):
        m = m + hist_all[t, pl.ds(j * L, L)]
      rm = lax.rev(m, (0,))
      cs = plsc.cumsum(rm)
      tot = jnp.sum(m)
      ge = (r + cs) >= k_rank
      ffsv = plsc.all_reduce_ffs(ge)
      i = _xt_i(ffsv, 0)
      lane = (L - 1) - i
      bucket = j * L + lane
      mb = jnp.sum(jnp.where(iota == i, rm, zeros_i))
      csi = jnp.sum(jnp.where(iota == i, cs, zeros_i))
      cum_before = r + csi - mb
      hit = (r < k_rank) & ((r + tot) >= k_rank)
      b_star = jnp.where(hit, bucket, b_star)
      rem = jnp.where(hit, k_rank - cum_before, rem)
      return (r + tot, b_star, rem)
    r, b_star, rem = lax.fori_loop(
        0, NB // L, sbody,
        (jnp.int32(0), jnp.int32(-1), jnp.int32(0)))
    return b_star, rem, r

  def publish_hist():
    pltpu.sync_copy(hist, sh_hist.at[tid])
    plsc.subcore_barrier()

  def read_ctl():
    pltpu.sync_copy(sh_ctli, ctli)
    v = ctli[pl.ds(0, L)]
    return _xt_i(v, 0), _xt_i(v, 1), _xt_i(v, 2), _xt_i(v, 3)

  def write_ctl(p, k, ok, nv):
    v = (jnp.where(iota == 0, p, 0) + jnp.where(iota == 1, k, 0)
         + jnp.where(iota == 2, ok, 0) + jnp.where(iota == 3, nv, 0))
    ctli[pl.ds(0, L)] = v.astype(jnp.int32)
    pltpu.sync_copy(ctli, sh_ctli)

  publish_hist()

  @pl.when(tid == 0)
  def _():
    pltpu.sync_copy(sh_hist, hist_all)
    b1, rem1, nv = scan_hist(jnp.int32(TOPK))
    ok = (b1 >= 0).astype(jnp.int32)
    write_ctl(lax.shift_left(jnp.maximum(b1, 0), 20), rem1, ok, nv)

  zero_hist()
  plsc.subcore_barrier()

  for shift in (10, 0):
    p_run, k_run, ok_run, nv_run = read_ctl()
    hi = shift + 10

    @plsc.parallel_loop(0, NUM_CLASSES * PP, L, unroll=4)
    def _(o, shift=shift, hi=hi, p_run=p_run):
      s = scores[pl.ds(o, L)]
      bits = lax.bitcast_convert_type(s, jnp.int32)
      m = (s > thr) & (lax.shift_right_logical(bits, hi)
                       == lax.shift_right_logical(p_run, hi))
      bkt = lax.shift_right_logical(bits, shift) & (NB - 1)
      plsc.addupdate_scatter(hist, [bkt], ones_i, mask=m)

    publish_hist()

    @pl.when(tid == 0)
    def _(shift=shift, p_run=p_run, k_run=k_run, ok_run=ok_run,
          nv_run=nv_run):
      pltpu.sync_copy(sh_hist, hist_all)
      b, rem, _tot = scan_hist(k_run)
      ok = ok_run * (b >= 0).astype(jnp.int32)
      p_new = p_run | lax.shift_left(jnp.maximum(b, 0), shift)
      write_ctl(p_new, rem, ok, nv_run)
      if shift == 0:
        strict = jnp.where((nv_run < TOPK) | (ok == 0),
                           jnp.int32(1), jnp.int32(0))
        t_bits = jnp.zeros((L,), jnp.int32) + p_new
        t_vec = lax.bitcast_convert_type(t_bits, jnp.float32)
        t_vec = jnp.where((jnp.zeros((L,), jnp.int32) + strict) == 1,
                          jnp.full((L,), THR, jnp.float32), t_vec)
        ctlf[pl.ds(0, L)] = t_vec
        pltpu.sync_copy(ctlf, sh_ctlf)
        v = (jnp.where(iota == 0, p_new, 0) + jnp.where(iota == 1, rem, 0)
             + jnp.where(iota == 2, strict, 0)
             + jnp.where(iota == 3, nv_run, 0))
        ctli[pl.ds(0, L)] = v.astype(jnp.int32)
        pltpu.sync_copy(ctli, sh_ctli)

    if shift != 0:
      zero_hist()
    plsc.subcore_barrier()

  _p_fin, _k_fin, strict_fin, _nv = read_ctl()
  pltpu.sync_copy(sh_ctlf, ctlf)
  t_vec = ctlf[pl.ds(0, L)]
  nonstrict = (jnp.zeros((L,), jnp.int32) + strict_fin) == 0

  zf = jnp.zeros((L,), jnp.float32)

  @plsc.parallel_loop(0, PP, L, unroll=2,
                      carry=(zf, zf, zf, zf, zf, zf, zf, zf, zf, zf))
  def acc(base, carry):
    nX, SxX, SyX, SxyX, SxxX, nY, SxY, SyY, SxyY, SxxY = carry
    kept = jnp.zeros((L,), jnp.bool_)
    xv = None
    yv = None
    for c in range(NUM_CLASSES):
      sv = scores[pl.ds(c * PP + base, L)]
      selc = (sv > t_vec) | (nonstrict & (sv == t_vec))
      kept = kept | selc
      if c == 3:
        xv = sv
      if c == 5:
        yv = sv
    gt = xv > yv
    mx = kept & gt
    my = kept & (~gt)
    x0 = xv - 0.5
    y0 = yv - 0.5
    wx = jnp.where(mx, 1.0, 0.0).astype(jnp.float32)
    wy = jnp.where(my, 1.0, 0.0).astype(jnp.float32)
    xy = x0 * y0
    xx = x0 * x0
    return (nX + wx, SxX + x0 * wx, SyX + y0 * wx, SxyX + xy * wx,
            SxxX + xx * wx, nY + wy, SxY + x0 * wy, SyY + y0 * wy,
            SxyY + xy * wy, SxxY + xx * wy)
  sums = zf
  for i, a in enumerate(acc):
    sums = sums + jnp.where(iota == i, jnp.sum(a), jnp.float32(0.0))
  fvec[pl.ds(0, L)] = sums
  pltpu.sync_copy(fvec, sh_sums.at[tid])
  plsc.subcore_barrier()

  @pl.when((tid == 0) & (cid == 0))
  def _():
    pltpu.sync_copy(sh_sums, sums_all)
    tv = zf
    for t in range(NS):
      tv = tv + sums_all[t, pl.ds(0, L)]

    def xt(lane):
      return jnp.zeros((L,), jnp.float32) + jnp.sum(
          jnp.where(iota == lane, tv, zf))

    nX, SxX, SyX, SxyX, SxxX = xt(0), xt(1), xt(2), xt(3), xt(4)
    nY, SxY, SyY, SxyY, SxxY = xt(5), xt(6), xt(7), xt(8), xt(9)

    def slope(n, sx, sy, sxy, sxx):
      return (sxy - sx * sy / n) / (sxx - sx * sx / n)

    sx_ = slope(nX, SxX, SyX, SxyX, SxxX)
    sy_ = slope(nY, SxY, SyY, SxyY, SxxY)
    t = jnp.abs((sy_ - sx_) / (1.0 + sy_ * sx_))
    tan3pi8 = jnp.float32(2.414213562373095)
    tanpi8 = jnp.float32(0.4142135623730951)
    big = t > tan3pi8
    mid = (t > tanpi8) & (~big)
    u = jnp.where(big, -1.0 / t, jnp.where(mid, (t - 1.0) / (t + 1.0), t))
    z = u * u
    poly = ((jnp.float32(8.05374449538e-2) * z
             - jnp.float32(1.38776856032e-1)) * z
            + jnp.float32(1.99777106478e-1)) * z - jnp.float32(
                3.33329491539e-1)
    at = poly * z * u + u
    at = at + jnp.where(big, jnp.float32(1.5707963267948966),
                        jnp.where(mid, jnp.float32(0.7853981633974483),
                                  jnp.float32(0.0)))
    sca = at * jnp.float32(57.29577951308232)
    nkept = nX + nY
    res = jnp.where(nkept > 0.5, sca, jnp.full((L,), jnp.nan, jnp.float32))
    fvec[pl.ds(0, L)] = res
    pltpu.sync_copy(fvec.at[pl.ds(0, L)], out_hbm)


def kernel(cls_score):
  x = cls_score.reshape(2, NUM_CLASSES, NS * NCHUNK, CHUNK)
  out = _sc_kernel(x)
  return out[0]
